# baseline (speedup 1.0000x reference)
import functools

import jax
import jax.numpy as jnp
from jax import lax
from jax.experimental import pallas as pl
from jax.experimental.pallas import tpu as pltpu

T = 2048
C = 768
E = 8
H = 4 * C
TM = 128
NT = T // TM + 8
TPAD = NT * TM
TR = 256


def _ln(z, g, b):
    m = jnp.mean(z, axis=-1, keepdims=True)
    v = jnp.mean((z - m) ** 2, axis=-1, keepdims=True)
    return (z - m) * lax.rsqrt(v + 1e-5) * g + b


def _prologue_body(x_ref, xp_ref, wrkvs_ref, wo_ref, wroute_ref, confb_ref,
                   cap_ref, ln1g_ref, ln1b_ref, ln2g_ref, ln2b_ref,
                   x1_ref, h_ref, st_ref, v_ref, win_ref, cost_ref, diff_ref,
                   aff_ref, scale_ref):
    i = pl.program_id(0)
    g1, b1 = ln1g_ref[...], ln1b_ref[...]
    h1 = _ln(x_ref[...], g1, b1)
    h1s = _ln(xp_ref[...], g1, b1)
    row = lax.broadcasted_iota(jnp.int32, h1s.shape, 0) + i * TR
    h1s = jnp.where(row == 0, 0.0, h1s)
    mix = 0.5 * (h1 + h1s)
    rkvs = jnp.dot(mix, wrkvs_ref[...], preferred_element_type=jnp.float32)
    r = jax.nn.sigmoid(rkvs[:, 0:C])
    k = rkvs[:, C:2 * C]
    v = rkvs[:, 2 * C:3 * C]
    st = rkvs[:, 3 * C:4 * C]
    att = jnp.dot(r * k * v, wo_ref[...], preferred_element_type=jnp.float32)
    x1 = x_ref[...] + att
    h = _ln(x1, ln2g_ref[...], ln2b_ref[...])
    route = jnp.dot(h, wroute_ref[...], preferred_element_type=jnp.float32)
    conf = jax.nn.sigmoid(route[:, 0:E] + confb_ref[...])
    diff = jax.nn.sigmoid(route[:, E:E + 1])
    aff = route[:, E + 1:E + 1 + E]
    bids = conf * cap_ref[...] + 0.01 * aff
    maxb = jnp.max(bids, axis=-1, keepdims=True)
    eio = lax.broadcasted_iota(jnp.int32, bids.shape, 1)
    win = jnp.min(jnp.where(bids >= maxb, eio, E), axis=-1, keepdims=True)
    wb = jnp.sum(jnp.where(eio == win, conf, 0.0), axis=-1, keepdims=True)
    x1_ref[...] = x1
    h_ref[...] = h
    st_ref[...] = st
    v_ref[...] = v
    win_ref[...] = win
    cost_ref[...] = maxb * diff
    diff_ref[...] = diff
    aff_ref[...] = aff
    scale_ref[...] = wb / (wb + 1e-6)


def _prologue(x2d, xp2d, wrkvs, wo, wroute, confb, cap, g1, b1, g2, b2):
    rows = lambda i: (i, 0)
    whole = lambda i: (0, 0)
    f32 = jnp.float32
    return pl.pallas_call(
        _prologue_body,
        grid=(T // TR,),
        in_specs=[
            pl.BlockSpec((TR, C), rows),
            pl.BlockSpec((TR, C), rows),
            pl.BlockSpec((C, 4 * C), whole),
            pl.BlockSpec((C, C), whole),
            pl.BlockSpec((C, 2 * E + 1), whole),
            pl.BlockSpec((1, E), whole),
            pl.BlockSpec((1, E), whole),
            pl.BlockSpec((1, C), whole),
            pl.BlockSpec((1, C), whole),
            pl.BlockSpec((1, C), whole),
            pl.BlockSpec((1, C), whole),
        ],
        out_specs=[
            pl.BlockSpec((TR, C), rows),
            pl.BlockSpec((TR, C), rows),
            pl.BlockSpec((TR, C), rows),
            pl.BlockSpec((TR, C), rows),
            pl.BlockSpec((TR, 1), rows),
            pl.BlockSpec((TR, 1), rows),
            pl.BlockSpec((TR, 1), rows),
            pl.BlockSpec((TR, E), rows),
            pl.BlockSpec((TR, 1), rows),
        ],
        out_shape=[
            jax.ShapeDtypeStruct((T, C), f32),
            jax.ShapeDtypeStruct((T, C), f32),
            jax.ShapeDtypeStruct((T, C), f32),
            jax.ShapeDtypeStruct((T, C), f32),
            jax.ShapeDtypeStruct((T, 1), jnp.int32),
            jax.ShapeDtypeStruct((T, 1), f32),
            jax.ShapeDtypeStruct((T, 1), f32),
            jax.ShapeDtypeStruct((T, E), f32),
            jax.ShapeDtypeStruct((T, 1), f32),
        ],
    )(x2d, xp2d, wrkvs, wo, wroute, confb, cap, g1, b1, g2, b2)


def _ffn_body(te_ref, h_ref, st_ref, x1_ref, sc_ref, vm_ref,
              w1_ref, b1_ref, w2_ref, b2_ref, ws1_ref, wrec_ref,
              out_ref, rec_ref):
    i = pl.program_id(0)
    e = te_ref[i]
    h = h_ref[...]
    base = jnp.dot(h, w1_ref[0], preferred_element_type=jnp.float32) + b1_ref[0]

    @pl.when(i == 0)
    def _init():
        rec_ref[...] = jnp.zeros_like(rec_ref)

    @pl.when(e == E - 1)
    def _last_expert():
        st = st_ref[...]
        hid = jax.nn.relu(
            base + jnp.dot(st, ws1_ref[...], preferred_element_type=jnp.float32))
        out = jnp.dot(hid, w2_ref[0], preferred_element_type=jnp.float32) + b2_ref[0]
        out_ref[...] = x1_ref[...] + out * sc_ref[...]
        r = jnp.dot(h, wrec_ref[...], preferred_element_type=jnp.float32) - st
        rec_ref[...] += jnp.sum(
            jnp.sum(r * r, axis=-1, keepdims=True) * vm_ref[...]).reshape(1, 1)

    @pl.when(e != E - 1)
    def _ffn_expert():
        hr = jax.nn.relu(base)
        out = jnp.dot(hr * hr, w2_ref[0], preferred_element_type=jnp.float32) + b2_ref[0]
        out_ref[...] = x1_ref[...] + out * sc_ref[...]


def _ffn(tile_expert, h_s, st_s, x1_s, sc_s, vm_s, w1, b1e, w2, b2e, ws1, wrec):
    rows = lambda i, te: (i, 0)
    byexp3 = lambda i, te: (te[i], 0, 0)
    whole = lambda i, te: (0, 0)
    f32 = jnp.float32
    grid_spec = pltpu.PrefetchScalarGridSpec(
        num_scalar_prefetch=1,
        grid=(NT,),
        in_specs=[
            pl.BlockSpec((TM, C), rows),
            pl.BlockSpec((TM, C), rows),
            pl.BlockSpec((TM, C), rows),
            pl.BlockSpec((TM, 1), rows),
            pl.BlockSpec((TM, 1), rows),
            pl.BlockSpec((1, C, H), byexp3),
            pl.BlockSpec((1, 1, H), byexp3),
            pl.BlockSpec((1, H, C), byexp3),
            pl.BlockSpec((1, 1, C), byexp3),
            pl.BlockSpec((C, H), whole),
            pl.BlockSpec((C, C), whole),
        ],
        out_specs=[
            pl.BlockSpec((TM, C), rows),
            pl.BlockSpec((1, 1), whole),
        ],
    )
    return pl.pallas_call(
        _ffn_body,
        grid_spec=grid_spec,
        out_shape=[
            jax.ShapeDtypeStruct((TPAD, C), f32),
            jax.ShapeDtypeStruct((1, 1), f32),
        ],
    )(tile_expert, h_s, st_s, x1_s, sc_s, vm_s, w1, b1e, w2, b2e, ws1, wrec)


def kernel(x, v_first, capital_shares, ln1_g, ln1_b, ln2_g, ln2_b, Wr, Wk, Wv,
           Wo, Ws, W1, b1, W2, b2, Ws1, Wrec, conf_w, conf_b, Wd, Wa):
    f32 = jnp.float32
    x2d = x.reshape(T, C)
    xp2d = jnp.concatenate([jnp.zeros((1, C), f32), x2d[:-1]], axis=0)
    wrkvs = jnp.concatenate([Wr, Wk, Wv, Ws], axis=1)
    wroute = jnp.concatenate([conf_w.T, Wd, Wa], axis=1)

    (x1, h, st, v, win2, cost2, diff2, aff, scale2) = _prologue(
        x2d, xp2d, wrkvs, Wo, wroute, conf_b.reshape(1, E),
        capital_shares.reshape(1, E), ln1_g.reshape(1, C), ln1_b.reshape(1, C),
        ln2_g.reshape(1, C), ln2_b.reshape(1, C))

    winners = win2[:, 0]
    counts = jnp.sum((winners[:, None] == jnp.arange(E)[None, :]).astype(jnp.int32), axis=0)
    tiles_e = (counts + TM - 1) // TM
    cum_tiles = jnp.cumsum(tiles_e)
    pstart = (cum_tiles - tiles_e) * TM
    offs = jnp.cumsum(counts) - counts
    ti = jnp.arange(NT)
    tile_expert = jnp.minimum(
        jnp.sum((ti[:, None] >= cum_tiles[None, :]).astype(jnp.int32), axis=1),
        E - 1).astype(jnp.int32)
    sort_idx = jnp.argsort(winners)
    rank = jnp.argsort(sort_idx)
    inv_perm = (pstart[winners] + (rank - offs[winners])).astype(jnp.int32)
    qi = jnp.arange(TPAD)
    qe = tile_expert[qi // TM]
    j = qi - pstart[qe]
    valid = j < counts[qe]
    src_row = jnp.where(
        valid, sort_idx[jnp.clip(offs[qe] + j, 0, T - 1)], 0).astype(jnp.int32)
    vm_s = valid.astype(f32)[:, None]
    sc_s = scale2[src_row]

    h_s = jnp.take(h, src_row, axis=0)
    st_s = jnp.take(st, src_row, axis=0)
    x1_s = jnp.take(x1, src_row, axis=0)

    ffn_out, rec_sum = _ffn(tile_expert, h_s, st_s, x1_s, sc_s, vm_s,
                            W1, b1.reshape(E, 1, H), W2, b2.reshape(E, 1, C),
                            Ws1, Wrec)

    x_out = jnp.take(ffn_out, inv_perm, axis=0)

    cnt7 = counts[E - 1]
    recon = jnp.where(cnt7 > 0, rec_sum[0, 0] / (cnt7 * C).astype(f32), 0.0)

    return (x_out.reshape(1, T, C), v.reshape(1, T, C), winners.reshape(1, T),
            cost2[:, 0].reshape(1, T), diff2.reshape(1, T, 1),
            aff.reshape(1, T, E), recon)


# speedup vs baseline: 1.1184x; 1.1184x over previous
import functools

import jax
import jax.numpy as jnp
from jax import lax
from jax.experimental import pallas as pl
from jax.experimental.pallas import tpu as pltpu
from jax.experimental.pallas import tpu_sc as plsc

T = 2048
C = 768
E = 8
H = 4 * C
TM = 128
NT = T // TM + 8
TPAD = NT * TM
TR = 256


_NC, _NS = 2, 16
_NW = _NC * _NS


def _sc_mesh():
    return plsc.VectorSubcoreMesh(core_axis_name="c", subcore_axis_name="s",
                                  num_cores=_NC, num_subcores=_NS)


def _row_gather3(h, st, x1, idx):
    bpw = TPAD // _NW
    f32 = jnp.float32
    out_t = [jax.ShapeDtypeStruct((TPAD, C), f32)] * 3

    @functools.partial(
        pl.kernel, out_type=out_t, mesh=_sc_mesh(),
        scratch_types=[pltpu.VMEM((bpw,), jnp.int32),
                       pltpu.VMEM((bpw, C), f32),
                       pltpu.SemaphoreType.DMA])
    def k(h_hbm, st_hbm, x1_hbm, idx_hbm, oh_hbm, ost_hbm, ox1_hbm,
          idx_v, rows_v, sem):
        wid = lax.axis_index("s") * _NC + lax.axis_index("c")
        base = wid * bpw
        pltpu.sync_copy(idx_hbm.at[pl.ds(base, bpw)], idx_v)
        pltpu.async_copy(h_hbm.at[idx_v], rows_v, sem).wait()
        pltpu.sync_copy(rows_v, oh_hbm.at[pl.ds(base, bpw)])
        pltpu.async_copy(st_hbm.at[idx_v], rows_v, sem).wait()
        pltpu.sync_copy(rows_v, ost_hbm.at[pl.ds(base, bpw)])
        pltpu.async_copy(x1_hbm.at[idx_v], rows_v, sem).wait()
        pltpu.sync_copy(rows_v, ox1_hbm.at[pl.ds(base, bpw)])

    return k(h, st, x1, idx)


def _row_gather1(src, idx, n_out):
    bpw = n_out // _NW
    f32 = jnp.float32

    @functools.partial(
        pl.kernel, out_type=jax.ShapeDtypeStruct((n_out, C), f32),
        mesh=_sc_mesh(),
        scratch_types=[pltpu.VMEM((bpw,), jnp.int32),
                       pltpu.VMEM((bpw, C), f32),
                       pltpu.SemaphoreType.DMA])
    def k(src_hbm, idx_hbm, out_hbm, idx_v, rows_v, sem):
        wid = lax.axis_index("s") * _NC + lax.axis_index("c")
        base = wid * bpw
        pltpu.sync_copy(idx_hbm.at[pl.ds(base, bpw)], idx_v)
        pltpu.async_copy(src_hbm.at[idx_v], rows_v, sem).wait()
        pltpu.sync_copy(rows_v, out_hbm.at[pl.ds(base, bpw)])

    return k(src, idx)


def _ln(z, g, b):
    m = jnp.mean(z, axis=-1, keepdims=True)
    v = jnp.mean((z - m) ** 2, axis=-1, keepdims=True)
    return (z - m) * lax.rsqrt(v + 1e-5) * g + b


def _prologue_body(x_ref, xp_ref, wrkvs_ref, wo_ref, wroute_ref, confb_ref,
                   cap_ref, ln1g_ref, ln1b_ref, ln2g_ref, ln2b_ref,
                   x1_ref, h_ref, st_ref, v_ref, win_ref, cost_ref, diff_ref,
                   aff_ref, scale_ref):
    i = pl.program_id(0)
    g1, b1 = ln1g_ref[...], ln1b_ref[...]
    h1 = _ln(x_ref[...], g1, b1)
    h1s = _ln(xp_ref[...], g1, b1)
    row = lax.broadcasted_iota(jnp.int32, h1s.shape, 0) + i * TR
    h1s = jnp.where(row == 0, 0.0, h1s)
    mix = 0.5 * (h1 + h1s)
    rkvs = jnp.dot(mix, wrkvs_ref[...], preferred_element_type=jnp.float32)
    r = jax.nn.sigmoid(rkvs[:, 0:C])
    k = rkvs[:, C:2 * C]
    v = rkvs[:, 2 * C:3 * C]
    st = rkvs[:, 3 * C:4 * C]
    att = jnp.dot(r * k * v, wo_ref[...], preferred_element_type=jnp.float32)
    x1 = x_ref[...] + att
    h = _ln(x1, ln2g_ref[...], ln2b_ref[...])
    route = jnp.dot(h, wroute_ref[...], preferred_element_type=jnp.float32)
    conf = jax.nn.sigmoid(route[:, 0:E] + confb_ref[...])
    diff = jax.nn.sigmoid(route[:, E:E + 1])
    aff = route[:, E + 1:E + 1 + E]
    bids = conf * cap_ref[...] + 0.01 * aff
    maxb = jnp.max(bids, axis=-1, keepdims=True)
    eio = lax.broadcasted_iota(jnp.int32, bids.shape, 1)
    win = jnp.min(jnp.where(bids >= maxb, eio, E), axis=-1, keepdims=True)
    wb = jnp.sum(jnp.where(eio == win, conf, 0.0), axis=-1, keepdims=True)
    x1_ref[...] = x1
    h_ref[...] = h
    st_ref[...] = st
    v_ref[...] = v
    win_ref[...] = win
    cost_ref[...] = maxb * diff
    diff_ref[...] = diff
    aff_ref[...] = aff
    scale_ref[...] = wb / (wb + 1e-6)


def _prologue(x2d, xp2d, wrkvs, wo, wroute, confb, cap, g1, b1, g2, b2):
    rows = lambda i: (i, 0)
    whole = lambda i: (0, 0)
    f32 = jnp.float32
    return pl.pallas_call(
        _prologue_body,
        grid=(T // TR,),
        in_specs=[
            pl.BlockSpec((TR, C), rows),
            pl.BlockSpec((TR, C), rows),
            pl.BlockSpec((C, 4 * C), whole),
            pl.BlockSpec((C, C), whole),
            pl.BlockSpec((C, 2 * E + 1), whole),
            pl.BlockSpec((1, E), whole),
            pl.BlockSpec((1, E), whole),
            pl.BlockSpec((1, C), whole),
            pl.BlockSpec((1, C), whole),
            pl.BlockSpec((1, C), whole),
            pl.BlockSpec((1, C), whole),
        ],
        out_specs=[
            pl.BlockSpec((TR, C), rows),
            pl.BlockSpec((TR, C), rows),
            pl.BlockSpec((TR, C), rows),
            pl.BlockSpec((TR, C), rows),
            pl.BlockSpec((TR, 1), rows),
            pl.BlockSpec((TR, 1), rows),
            pl.BlockSpec((TR, 1), rows),
            pl.BlockSpec((TR, E), rows),
            pl.BlockSpec((TR, 1), rows),
        ],
        out_shape=[
            jax.ShapeDtypeStruct((T, C), f32),
            jax.ShapeDtypeStruct((T, C), f32),
            jax.ShapeDtypeStruct((T, C), f32),
            jax.ShapeDtypeStruct((T, C), f32),
            jax.ShapeDtypeStruct((T, 1), jnp.int32),
            jax.ShapeDtypeStruct((T, 1), f32),
            jax.ShapeDtypeStruct((T, 1), f32),
            jax.ShapeDtypeStruct((T, E), f32),
            jax.ShapeDtypeStruct((T, 1), f32),
        ],
    )(x2d, xp2d, wrkvs, wo, wroute, confb, cap, g1, b1, g2, b2)


def _ffn_body(te_ref, h_ref, st_ref, x1_ref, sc_ref, vm_ref,
              w1_ref, b1_ref, w2_ref, b2_ref, ws1_ref, wrec_ref,
              out_ref, rec_ref):
    i = pl.program_id(0)
    e = te_ref[i]
    h = h_ref[...]
    base = jnp.dot(h, w1_ref[0], preferred_element_type=jnp.float32) + b1_ref[0]

    @pl.when(i == 0)
    def _init():
        rec_ref[...] = jnp.zeros_like(rec_ref)

    @pl.when(e == E - 1)
    def _last_expert():
        st = st_ref[...]
        hid = jax.nn.relu(
            base + jnp.dot(st, ws1_ref[...], preferred_element_type=jnp.float32))
        out = jnp.dot(hid, w2_ref[0], preferred_element_type=jnp.float32) + b2_ref[0]
        out_ref[...] = x1_ref[...] + out * sc_ref[...]
        r = jnp.dot(h, wrec_ref[...], preferred_element_type=jnp.float32) - st
        rec_ref[...] += jnp.sum(
            jnp.sum(r * r, axis=-1, keepdims=True) * vm_ref[...]).reshape(1, 1)

    @pl.when(e != E - 1)
    def _ffn_expert():
        hr = jax.nn.relu(base)
        out = jnp.dot(hr * hr, w2_ref[0], preferred_element_type=jnp.float32) + b2_ref[0]
        out_ref[...] = x1_ref[...] + out * sc_ref[...]


def _ffn(tile_expert, h_s, st_s, x1_s, sc_s, vm_s, w1, b1e, w2, b2e, ws1, wrec):
    rows = lambda i, te: (i, 0)
    byexp3 = lambda i, te: (te[i], 0, 0)
    whole = lambda i, te: (0, 0)
    f32 = jnp.float32
    grid_spec = pltpu.PrefetchScalarGridSpec(
        num_scalar_prefetch=1,
        grid=(NT,),
        in_specs=[
            pl.BlockSpec((TM, C), rows),
            pl.BlockSpec((TM, C), rows),
            pl.BlockSpec((TM, C), rows),
            pl.BlockSpec((TM, 1), rows),
            pl.BlockSpec((TM, 1), rows),
            pl.BlockSpec((1, C, H), byexp3),
            pl.BlockSpec((1, 1, H), byexp3),
            pl.BlockSpec((1, H, C), byexp3),
            pl.BlockSpec((1, 1, C), byexp3),
            pl.BlockSpec((C, H), whole),
            pl.BlockSpec((C, C), whole),
        ],
        out_specs=[
            pl.BlockSpec((TM, C), rows),
            pl.BlockSpec((1, 1), whole),
        ],
    )
    return pl.pallas_call(
        _ffn_body,
        grid_spec=grid_spec,
        out_shape=[
            jax.ShapeDtypeStruct((TPAD, C), f32),
            jax.ShapeDtypeStruct((1, 1), f32),
        ],
    )(tile_expert, h_s, st_s, x1_s, sc_s, vm_s, w1, b1e, w2, b2e, ws1, wrec)


def kernel(x, v_first, capital_shares, ln1_g, ln1_b, ln2_g, ln2_b, Wr, Wk, Wv,
           Wo, Ws, W1, b1, W2, b2, Ws1, Wrec, conf_w, conf_b, Wd, Wa):
    f32 = jnp.float32
    x2d = x.reshape(T, C)
    xp2d = jnp.concatenate([jnp.zeros((1, C), f32), x2d[:-1]], axis=0)
    wrkvs = jnp.concatenate([Wr, Wk, Wv, Ws], axis=1)
    wroute = jnp.concatenate([conf_w.T, Wd, Wa], axis=1)

    (x1, h, st, v, win2, cost2, diff2, aff, scale2) = _prologue(
        x2d, xp2d, wrkvs, Wo, wroute, conf_b.reshape(1, E),
        capital_shares.reshape(1, E), ln1_g.reshape(1, C), ln1_b.reshape(1, C),
        ln2_g.reshape(1, C), ln2_b.reshape(1, C))

    winners = win2[:, 0]
    counts = jnp.sum((winners[:, None] == jnp.arange(E)[None, :]).astype(jnp.int32), axis=0)
    tiles_e = (counts + TM - 1) // TM
    cum_tiles = jnp.cumsum(tiles_e)
    pstart = (cum_tiles - tiles_e) * TM
    offs = jnp.cumsum(counts) - counts
    ti = jnp.arange(NT)
    tile_expert = jnp.minimum(
        jnp.sum((ti[:, None] >= cum_tiles[None, :]).astype(jnp.int32), axis=1),
        E - 1).astype(jnp.int32)
    sort_idx = jnp.argsort(winners)
    rank = jnp.argsort(sort_idx)
    inv_perm = (pstart[winners] + (rank - offs[winners])).astype(jnp.int32)
    qi = jnp.arange(TPAD)
    qe = tile_expert[qi // TM]
    j = qi - pstart[qe]
    valid = j < counts[qe]
    src_row = jnp.where(
        valid, sort_idx[jnp.clip(offs[qe] + j, 0, T - 1)], 0).astype(jnp.int32)
    vm_s = valid.astype(f32)[:, None]
    sc_s = scale2[src_row]

    h_s, st_s, x1_s = _row_gather3(h, st, x1, src_row)

    ffn_out, rec_sum = _ffn(tile_expert, h_s, st_s, x1_s, sc_s, vm_s,
                            W1, b1.reshape(E, 1, H), W2, b2.reshape(E, 1, C),
                            Ws1, Wrec)

    x_out = _row_gather1(ffn_out, inv_perm, T)

    cnt7 = counts[E - 1]
    recon = jnp.where(cnt7 > 0, rec_sum[0, 0] / (cnt7 * C).astype(f32), 0.0)

    return (x_out.reshape(1, T, C), v.reshape(1, T, C), winners.reshape(1, T),
            cost2[:, 0].reshape(1, T), diff2.reshape(1, T, 1),
            aff.reshape(1, T, E), recon)
